# baseline (speedup 1.0000x reference)
import functools

import jax
import jax.numpy as jnp
from jax import lax
from jax.experimental import pallas as pl
from jax.experimental.pallas import tpu as pltpu
from jax.experimental.pallas import tpu_sc as plsc

_TAU = 2048


def _build_sc_edges_kernel(B, E, H, L, NC):
    tail = _TAU * H
    out_e = E + tail
    HALF = E // 2
    NPIPE = 4
    C = HALF // NPIPE
    period = H * L
    nper = tail // period
    assert tail % period == 0 and E % 4 == 0

    shift = 16
    mult = -(-(1 << shift) // H)
    for j in range(period):
        assert (j * mult) >> shift == j // H

    mesh = plsc.VectorSubcoreMesh(core_axis_name="c", subcore_axis_name="s")

    @functools.partial(
        pl.kernel,
        mesh=mesh,
        out_type=jax.ShapeDtypeStruct((B, 2, out_e), jnp.int32),
        scratch_types=[
            pltpu.VMEM((HALF,), jnp.int32),
            pltpu.VMEM((tail,), jnp.int32),
            pltpu.VMEM((B + H, L), jnp.int32),
            pltpu.SemaphoreType.DMA,
            pltpu.SemaphoreType.DMA,
            pltpu.SemaphoreType.DMA,
            pltpu.SemaphoreType.DMA,
            pltpu.SemaphoreType.DMA,
            pltpu.SemaphoreType.DMA,
        ],
    )
    def sc_k(e_hbm, params_hbm, eout_hbm, buf, tl, par_v,
             s0, s1, s2, s3, sem_p, sem_o):
        c = lax.axis_index("c")
        s = lax.axis_index("s")
        w = s * NC + c
        row = lax.div(w, 2)
        b = lax.div(row, 2)
        i = lax.rem(row, 2)
        half = lax.rem(w, 2)
        off = half * HALF
        do_tail = lax.rem(w, 2) == lax.rem(row, 2)
        sems = [s0, s1, s2, s3]

        @pl.when(do_tail)
        def _params():
            pltpu.async_copy(params_hbm, par_v, sem_p)

        for k in range(NPIPE):
            pltpu.async_copy(
                e_hbm.at[b, i, pl.ds(off + k * C, C)],
                buf.at[pl.ds(k * C, C)], sems[k]
            )

        for k in range(NPIPE):
            pltpu.make_async_copy(
                e_hbm.at[0, 0, pl.ds(0, C)], buf.at[pl.ds(k * C, C)], sems[k]
            ).wait()
            pltpu.async_copy(
                buf.at[pl.ds(k * C, C)],
                eout_hbm.at[b, i, pl.ds(off + k * C, C)], sem_o
            )

        @pl.when(do_tail)
        def _gen_tail():
            pltpu.make_async_copy(params_hbm, par_v, sem_p).wait()
            base_v = par_v[b]
            i_v = jnp.full((L,), i, jnp.int32)
            lanes = lax.broadcasted_iota(jnp.int32, (L,), 0)
            seeds = []
            for h in range(H):
                j = h * L + lanes
                t = (j * mult) >> shift
                r = j - t * H
                hop = par_v[B + H - 1]
                for hh in range(H - 2, -1, -1):
                    hop = jnp.where(r == hh, par_v[B + hh], hop)
                seeds.append(base_v + t - i_v * hop)

            def body(ci, carry):
                o = ci * period
                for h in range(H):
                    tl[pl.ds(o + h * L, L)] = carry[h]
                return tuple(v + L for v in carry)

            lax.fori_loop(0, nper, body, tuple(seeds))
            pltpu.async_copy(tl, eout_hbm.at[b, i, pl.ds(E, tail)], sem_o)

        for k in range(NPIPE):
            pltpu.make_async_copy(
                buf.at[pl.ds(k * C, C)], eout_hbm.at[0, 0, pl.ds(0, C)], sem_o
            ).wait()

        @pl.when(do_tail)
        def _tail_drain():
            pltpu.make_async_copy(
                tl, eout_hbm.at[0, 0, pl.ds(E, tail)], sem_o
            ).wait()

    return sc_k


def _build_tc_weights_kernel(B, E, H, wdtype):
    tail = _TAU * H
    out_e = E + tail

    def body(w_ref, o_ref, zbuf, sem_c, sem_z):
        zbuf[...] = jnp.zeros((tail,), wdtype)
        for b in range(B):
            pltpu.make_async_copy(
                w_ref.at[b, 0], o_ref.at[b, 0, pl.ds(0, E)], sem_c
            ).start()
            pltpu.make_async_copy(
                zbuf, o_ref.at[b, 0, pl.ds(E, tail)], sem_z
            ).start()
        for b in range(B):
            pltpu.make_async_copy(
                w_ref.at[b, 0], o_ref.at[b, 0, pl.ds(0, E)], sem_c
            ).wait()
            pltpu.make_async_copy(
                zbuf, o_ref.at[b, 0, pl.ds(E, tail)], sem_z
            ).wait()

    return pl.pallas_call(
        body,
        in_specs=[pl.BlockSpec(memory_space=pl.ANY)],
        out_specs=pl.BlockSpec(memory_space=pl.ANY),
        out_shape=jax.ShapeDtypeStruct((B, 1, out_e), wdtype),
        scratch_shapes=[
            pltpu.VMEM((tail,), wdtype),
            pltpu.SemaphoreType.DMA,
            pltpu.SemaphoreType.DMA,
        ],
    )


def kernel(nodes, edges, weights, T, taus, hops):
    del nodes
    B, _, E = edges.shape
    H = hops.shape[0]
    edtype = edges.dtype

    info = plsc.get_sparse_core_info()
    NC, L = info.num_cores, info.num_lanes

    base = T.astype(jnp.int32) + taus.astype(jnp.int32) - _TAU
    scal = jnp.concatenate([base, hops.astype(jnp.int32)])
    params = jnp.broadcast_to(scal[:, None], (B + H, L))

    sc_k = _build_sc_edges_kernel(B, E, H, L, NC)
    edges_out = sc_k(edges.astype(jnp.int32), params)
    weights_out = _build_tc_weights_kernel(B, E, H, weights.dtype)(weights)
    return edges_out.astype(edtype), weights_out


# speedup vs baseline: 3.2780x; 3.2780x over previous
import functools

import jax
import jax.numpy as jnp
from jax import lax
from jax.experimental import pallas as pl
from jax.experimental.pallas import tpu as pltpu
from jax.experimental.pallas import tpu_sc as plsc

_TAU = 2048


def _build_sc_edges_kernel(B, E, H, L, NC):
    tail = _TAU * H
    out_e = E + tail
    HALF = E // 2
    NPIPE = 4
    C = HALF // NPIPE
    period = H * L
    nper = tail // period
    assert tail % period == 0 and E % 4 == 0

    shift = 16
    mult = -(-(1 << shift) // H)
    for j in range(period):
        assert (j * mult) >> shift == j // H

    mesh = plsc.VectorSubcoreMesh(core_axis_name="c", subcore_axis_name="s")

    @functools.partial(
        pl.kernel,
        mesh=mesh,
        out_type=jax.ShapeDtypeStruct((B, 2, out_e), jnp.int32),
        scratch_types=[
            pltpu.VMEM((HALF,), jnp.int32),
            pltpu.VMEM((tail,), jnp.int32),
            pltpu.VMEM((B + H, L), jnp.int32),
            pltpu.SemaphoreType.DMA,
            pltpu.SemaphoreType.DMA,
            pltpu.SemaphoreType.DMA,
            pltpu.SemaphoreType.DMA,
            pltpu.SemaphoreType.DMA,
            pltpu.SemaphoreType.DMA,
        ],
    )
    def sc_k(e_hbm, params_hbm, eout_hbm, buf, tl, par_v,
             s0, s1, s2, s3, sem_p, sem_o):
        c = lax.axis_index("c")
        s = lax.axis_index("s")
        w = s * NC + c
        row = lax.div(w, 2)
        b = lax.div(row, 2)
        i = lax.rem(row, 2)
        half = lax.rem(w, 2)
        off = half * HALF
        do_tail = lax.rem(w, 2) == lax.rem(row, 2)
        sems = [s0, s1, s2, s3]

        @pl.when(do_tail)
        def _params():
            pltpu.async_copy(params_hbm, par_v, sem_p)

        for k in range(NPIPE):
            pltpu.async_copy(
                e_hbm.at[b, i, pl.ds(off + k * C, C)],
                buf.at[pl.ds(k * C, C)], sems[k]
            )

        for k in range(NPIPE):
            pltpu.make_async_copy(
                e_hbm.at[0, 0, pl.ds(0, C)], buf.at[pl.ds(k * C, C)], sems[k]
            ).wait()
            pltpu.async_copy(
                buf.at[pl.ds(k * C, C)],
                eout_hbm.at[b, i, pl.ds(off + k * C, C)], sem_o
            )

        @pl.when(do_tail)
        def _gen_tail():
            pltpu.make_async_copy(params_hbm, par_v, sem_p).wait()
            base_v = par_v[b]
            i_v = jnp.full((L,), i, jnp.int32)
            lanes = lax.broadcasted_iota(jnp.int32, (L,), 0)
            seeds = []
            for h in range(H):
                j = h * L + lanes
                t = (j * mult) >> shift
                r = j - t * H
                hop = par_v[B + H - 1]
                for hh in range(H - 2, -1, -1):
                    hop = jnp.where(r == hh, par_v[B + hh], hop)
                seeds.append(base_v + t - i_v * hop)

            def body(ci, carry):
                o = ci * period
                for h in range(H):
                    tl[pl.ds(o + h * L, L)] = carry[h]
                return tuple(v + L for v in carry)

            lax.fori_loop(0, nper, body, tuple(seeds))
            pltpu.async_copy(tl, eout_hbm.at[b, i, pl.ds(E, tail)], sem_o)

        for k in range(NPIPE):
            pltpu.make_async_copy(
                buf.at[pl.ds(k * C, C)], eout_hbm.at[0, 0, pl.ds(0, C)], sem_o
            ).wait()

        @pl.when(do_tail)
        def _tail_drain():
            pltpu.make_async_copy(
                tl, eout_hbm.at[0, 0, pl.ds(E, tail)], sem_o
            ).wait()

    return sc_k


def _build_tc_weights_kernel(B, E, H, wdtype):
    tail = _TAU * H
    out_e = E + tail

    def body(w_ref, o_ref):
        o_ref[:, :, pl.ds(0, E)] = w_ref[...]
        o_ref[:, :, pl.ds(E, tail)] = jnp.zeros((1, 1, tail), wdtype)

    return pl.pallas_call(
        body,
        grid=(B,),
        in_specs=[pl.BlockSpec((1, 1, E), lambda b: (b, 0, 0))],
        out_specs=pl.BlockSpec((1, 1, out_e), lambda b: (b, 0, 0)),
        out_shape=jax.ShapeDtypeStruct((B, 1, out_e), wdtype),
    )


def kernel(nodes, edges, weights, T, taus, hops):
    del nodes
    B, _, E = edges.shape
    H = hops.shape[0]
    edtype = edges.dtype

    info = plsc.get_sparse_core_info()
    NC, L = info.num_cores, info.num_lanes

    base = T.astype(jnp.int32) + taus.astype(jnp.int32) - _TAU
    scal = jnp.concatenate([base, hops.astype(jnp.int32)])
    params = jnp.broadcast_to(scal[:, None], (B + H, L))

    sc_k = _build_sc_edges_kernel(B, E, H, L, NC)
    edges_out = sc_k(edges.astype(jnp.int32), params)
    weights_out = _build_tc_weights_kernel(B, E, H, weights.dtype)(weights)
    return edges_out.astype(edtype), weights_out
